# baseline (speedup 1.0000x reference)
import jax
import jax.numpy as jnp
from jax.experimental import pallas as pl

N, D, E = 32768, 1024, 8
BC = 2048
HALF = (N * E) // 2

_ROT = ((13, 15, 26, 6), (17, 29, 16, 24))


def _rotl(v, d):
    return jnp.bitwise_or(jnp.left_shift(v, jnp.uint32(d)),
                          jnp.right_shift(v, jnp.uint32(32 - d)))


def _threefry2x32(ks, x0, x1):
    x0 = x0 + ks[0]
    x1 = x1 + ks[1]
    for r in range(5):
        for d in _ROT[r % 2]:
            x0 = x0 + x1
            x1 = _rotl(x1, d)
            x1 = jnp.bitwise_xor(x0, x1)
        x0 = x0 + ks[(r + 1) % 3]
        x1 = x1 + ks[(r + 2) % 3] + jnp.uint32(r + 1)
    return x0, x1


def _gumbel_block(pid):
    ks = (jnp.uint32(0), jnp.uint32(42),
          jnp.uint32(0) ^ jnp.uint32(42) ^ jnp.uint32(0x1BD11BDA))
    col = jax.lax.broadcasted_iota(jnp.uint32, (E, BC), 1)
    row = jax.lax.broadcasted_iota(jnp.uint32, (E, BC), 0)
    flat = jnp.uint32(pid) * jnp.uint32(E * BC) + col * jnp.uint32(E) + row
    o0, o1 = _threefry2x32(ks, jnp.zeros((E, BC), jnp.uint32), flat)
    bits = jnp.bitwise_xor(o0, o1)
    fbits = jnp.bitwise_or(jnp.right_shift(bits, jnp.uint32(9)),
                           jnp.uint32(0x3F800000))
    fl = jax.lax.bitcast_convert_type(fbits, jnp.float32) - jnp.float32(1.0)
    eps = jnp.float32(1e-9)
    u = jnp.maximum(eps, fl * (jnp.float32(1.0) - eps) + eps)
    return -jnp.log(-jnp.log(u))


def _gating_body(x_ref, w_ref, b_ref, u_ref, l_ref, bu_ref, bl_ref,
                 weights_ref, logits_ref):
    xb = x_ref[...]
    lt = jax.lax.dot_general(
        w_ref[...], xb, (((1,), (1,)), ((), ())),
        preferred_element_type=jnp.float32)
    lt = lt + b_ref[:, 0:1]
    row = jax.lax.broadcasted_iota(jnp.int32, (E, BC), 0)
    lt = lt + jnp.where(row == 0, bu_ref[0, 0] * u_ref[...], 0.0)
    lt = lt + jnp.where(row == 1, bl_ref[0, 0] * l_ref[...], 0.0)
    logits_ref[...] = lt.T
    z = lt + _gumbel_block(pl.program_id(0))
    z = z - jnp.max(z, axis=0, keepdims=True)
    e = jnp.exp(z)
    weights_ref[...] = (e / jnp.sum(e, axis=0, keepdims=True)).T


def kernel(x, uncertainty, logic_score, W, b, beta_uncertainty, beta_logic):
    b2 = jnp.broadcast_to(b.reshape(E, 1), (E, 128))
    ut = uncertainty.reshape(1, N)
    lt_ = logic_score.reshape(1, N)
    bu = beta_uncertainty.reshape(1, 1)
    bl = beta_logic.reshape(1, 1)

    grid = (N // BC,)
    weights_t, logits_t = pl.pallas_call(
        _gating_body,
        grid=grid,
        in_specs=[
            pl.BlockSpec((BC, D), lambda i: (i, 0)),
            pl.BlockSpec((E, D), lambda i: (0, 0)),
            pl.BlockSpec((E, 128), lambda i: (0, 0)),
            pl.BlockSpec((1, BC), lambda i: (0, i)),
            pl.BlockSpec((1, BC), lambda i: (0, i)),
            pl.BlockSpec((1, 1), lambda i: (0, 0)),
            pl.BlockSpec((1, 1), lambda i: (0, 0)),
        ],
        out_specs=[
            pl.BlockSpec((BC, E), lambda i: (i, 0)),
            pl.BlockSpec((BC, E), lambda i: (i, 0)),
        ],
        out_shape=[
            jax.ShapeDtypeStruct((N, E), jnp.float32),
            jax.ShapeDtypeStruct((N, E), jnp.float32),
        ],
    )(x, W, b2, ut, lt_, bu, bl)
    return weights_t, logits_t


# speedup vs baseline: 1.1462x; 1.1462x over previous
import jax
import jax.numpy as jnp
from jax import lax
from jax.experimental import pallas as pl
from jax.experimental.pallas import tpu as pltpu, tpu_sc as plsc

N, D, E = 32768, 1024, 8
BC = 2048
_ROT = ((13, 15, 26, 6), (17, 29, 16, 24))


def _rotl(v, d):
    return jnp.bitwise_or(jnp.left_shift(v, jnp.uint32(d)),
                          jnp.right_shift(v, jnp.uint32(32 - d)))


def _threefry2x32(ks, x0, x1):
    x0 = x0 + ks[0]
    x1 = x1 + ks[1]
    for r in range(5):
        for d in _ROT[r % 2]:
            x0 = x0 + x1
            x1 = _rotl(x1, d)
            x1 = jnp.bitwise_xor(x0, x1)
        x0 = x0 + ks[(r + 1) % 3]
        x1 = x1 + ks[(r + 2) % 3] + jnp.uint32(r + 1)
    return x0, x1


def _gumbel_block(pid):
    ks = (jnp.uint32(0), jnp.uint32(42),
          jnp.uint32(0) ^ jnp.uint32(42) ^ jnp.uint32(0x1BD11BDA))
    col = jax.lax.broadcasted_iota(jnp.uint32, (E, BC), 1)
    row = jax.lax.broadcasted_iota(jnp.uint32, (E, BC), 0)
    flat = jnp.uint32(pid) * jnp.uint32(E * BC) + col * jnp.uint32(E) + row
    o0, o1 = _threefry2x32(ks, jnp.zeros((E, BC), jnp.uint32), flat)
    bits = jnp.bitwise_xor(o0, o1)
    fbits = jnp.bitwise_or(jnp.right_shift(bits, jnp.uint32(9)),
                           jnp.uint32(0x3F800000))
    fl = jax.lax.bitcast_convert_type(fbits, jnp.float32) - jnp.float32(1.0)
    eps = jnp.float32(1e-9)
    u = jnp.maximum(eps, fl * (jnp.float32(1.0) - eps) + eps)
    return -jnp.log(-jnp.log(u))


def _gating_body(x_ref, w_ref, b_ref, u_ref, l_ref, bu_ref, bl_ref,
                 z_out_ref, logits_ref):
    xb = x_ref[...]
    lt = jax.lax.dot_general(
        w_ref[...], xb, (((1,), (1,)), ((), ())),
        preferred_element_type=jnp.float32)
    lt = lt + b_ref[:, 0:1]
    row = jax.lax.broadcasted_iota(jnp.int32, (E, BC), 0)
    lt = lt + jnp.where(row == 0, bu_ref[0, 0] * u_ref[...], 0.0)
    lt = lt + jnp.where(row == 1, bl_ref[0, 0] * l_ref[...], 0.0)
    logits_ref[...] = lt
    z_out_ref[...] = lt + _gumbel_block(pl.program_id(0))


_TPW = N // 32


def _sc_softmax_body(z_hbm, out_hbm, z_v, w_v):
    wid = lax.axis_index("s") * 2 + lax.axis_index("c")
    base = wid * _TPW
    pltpu.sync_copy(z_hbm.at[:, pl.ds(base, _TPW)], z_v)
    for c in range(_TPW // 16):
        sl = pl.ds(c * 16, 16)
        rows = [z_v[e, sl] for e in range(E)]
        m = rows[0]
        for e in range(1, E):
            m = jnp.maximum(m, rows[e])
        es = [jnp.exp(rows[e] - m) for e in range(E)]
        s = es[0]
        for e in range(1, E):
            s = s + es[e]
        for e in range(E):
            w_v[e, sl] = es[e] / s
    pltpu.sync_copy(w_v, out_hbm.at[:, pl.ds(base, _TPW)])


def _sc_softmax(z_t):
    mesh = plsc.VectorSubcoreMesh(core_axis_name="c", subcore_axis_name="s",
                                  num_cores=2, num_subcores=16)
    return pl.kernel(
        _sc_softmax_body,
        out_type=jax.ShapeDtypeStruct((E, N), jnp.float32),
        mesh=mesh,
        scratch_types=[
            pltpu.VMEM((E, _TPW), jnp.float32),
            pltpu.VMEM((E, _TPW), jnp.float32),
        ],
    )(z_t)


def kernel(x, uncertainty, logic_score, W, b, beta_uncertainty, beta_logic):
    b2 = jnp.broadcast_to(b.reshape(E, 1), (E, 128))
    ut = uncertainty.reshape(1, N)
    lt_ = logic_score.reshape(1, N)
    bu = beta_uncertainty.reshape(1, 1)
    bl = beta_logic.reshape(1, 1)

    grid = (N // BC,)
    z_t, logits_t = pl.pallas_call(
        _gating_body,
        grid=grid,
        in_specs=[
            pl.BlockSpec((BC, D), lambda i: (i, 0)),
            pl.BlockSpec((E, D), lambda i: (0, 0)),
            pl.BlockSpec((E, 128), lambda i: (0, 0)),
            pl.BlockSpec((1, BC), lambda i: (0, i)),
            pl.BlockSpec((1, BC), lambda i: (0, i)),
            pl.BlockSpec((1, 1), lambda i: (0, 0)),
            pl.BlockSpec((1, 1), lambda i: (0, 0)),
        ],
        out_specs=[
            pl.BlockSpec((E, BC), lambda i: (0, i)),
            pl.BlockSpec((E, BC), lambda i: (0, i)),
        ],
        out_shape=[
            jax.ShapeDtypeStruct((E, N), jnp.float32),
            jax.ShapeDtypeStruct((E, N), jnp.float32),
        ],
    )(x, W, b2, ut, lt_, bu, bl)
    weights_t = _sc_softmax(z_t)
    return weights_t.T, logits_t.T


# speedup vs baseline: 1.6728x; 1.4594x over previous
import jax
import jax.numpy as jnp
from jax.experimental import pallas as pl

N, D, E = 32768, 1024, 8
BC = 2048
_ROT = ((13, 15, 26, 6), (17, 29, 16, 24))


def _rotl(v, d):
    return jnp.bitwise_or(jnp.left_shift(v, jnp.uint32(d)),
                          jnp.right_shift(v, jnp.uint32(32 - d)))


def _threefry2x32(ks, x0, x1):
    x0 = x0 + ks[0]
    x1 = x1 + ks[1]
    for r in range(5):
        for d in _ROT[r % 2]:
            x0 = x0 + x1
            x1 = _rotl(x1, d)
            x1 = jnp.bitwise_xor(x0, x1)
        x0 = x0 + ks[(r + 1) % 3]
        x1 = x1 + ks[(r + 2) % 3] + jnp.uint32(r + 1)
    return x0, x1


def _gumbel_block(pid):
    ks = (jnp.uint32(0), jnp.uint32(42),
          jnp.uint32(0) ^ jnp.uint32(42) ^ jnp.uint32(0x1BD11BDA))
    col = jax.lax.broadcasted_iota(jnp.uint32, (E, BC), 1)
    row = jax.lax.broadcasted_iota(jnp.uint32, (E, BC), 0)
    flat = jnp.uint32(pid) * jnp.uint32(E * BC) + col * jnp.uint32(E) + row
    o0, o1 = _threefry2x32(ks, jnp.zeros((E, BC), jnp.uint32), flat)
    bits = jnp.bitwise_xor(o0, o1)
    fbits = jnp.bitwise_or(jnp.right_shift(bits, jnp.uint32(9)),
                           jnp.uint32(0x3F800000))
    fl = jax.lax.bitcast_convert_type(fbits, jnp.float32) - jnp.float32(1.0)
    eps = jnp.float32(1e-9)
    u = jnp.maximum(eps, fl * (jnp.float32(1.0) - eps) + eps)
    return -jnp.log(-jnp.log(u))


def _gating_body(x_ref, w_ref, b_ref, u_ref, l_ref, bu_ref, bl_ref,
                 weights_ref, logits_ref):
    xb = x_ref[...]
    lt = jax.lax.dot_general(
        w_ref[...], xb, (((1,), (1,)), ((), ())),
        preferred_element_type=jnp.float32)
    lt = lt + b_ref[:, 0:1]
    row = jax.lax.broadcasted_iota(jnp.int32, (E, BC), 0)
    lt = lt + jnp.where(row == 0, bu_ref[0, 0] * u_ref[...], 0.0)
    lt = lt + jnp.where(row == 1, bl_ref[0, 0] * l_ref[...], 0.0)
    logits_ref[...] = lt
    z = lt + _gumbel_block(pl.program_id(0))
    z = z - jnp.max(z, axis=0, keepdims=True)
    e = jnp.exp(z)
    weights_ref[...] = e / jnp.sum(e, axis=0, keepdims=True)


def kernel(x, uncertainty, logic_score, W, b, beta_uncertainty, beta_logic):
    b2 = jnp.broadcast_to(b.reshape(E, 1), (E, 128))
    ut = uncertainty.reshape(1, N)
    lt_ = logic_score.reshape(1, N)
    bu = beta_uncertainty.reshape(1, 1)
    bl = beta_logic.reshape(1, 1)

    grid = (N // BC,)
    weights_t, logits_t = pl.pallas_call(
        _gating_body,
        grid=grid,
        in_specs=[
            pl.BlockSpec((BC, D), lambda i: (i, 0)),
            pl.BlockSpec((E, D), lambda i: (0, 0)),
            pl.BlockSpec((E, 128), lambda i: (0, 0)),
            pl.BlockSpec((1, BC), lambda i: (0, i)),
            pl.BlockSpec((1, BC), lambda i: (0, i)),
            pl.BlockSpec((1, 1), lambda i: (0, 0)),
            pl.BlockSpec((1, 1), lambda i: (0, 0)),
        ],
        out_specs=[
            pl.BlockSpec((E, BC), lambda i: (0, i)),
            pl.BlockSpec((E, BC), lambda i: (0, i)),
        ],
        out_shape=[
            jax.ShapeDtypeStruct((E, N), jnp.float32),
            jax.ShapeDtypeStruct((E, N), jnp.float32),
        ],
    )(x, W, b2, ut, lt_, bu, bl)
    return weights_t.T, logits_t.T
